# Initial kernel scaffold; baseline (speedup 1.0000x reference)
import jax
import jax.numpy as jnp
from jax.experimental import pallas as pl


def kernel(x, embeddings):
    raise NotImplementedError("write your pallas kernel here")


# speedup vs baseline: 3.9107x; 3.9107x over previous
import functools

import jax
import jax.numpy as jnp
from jax.experimental import pallas as pl
from jax.experimental.pallas import tpu as pltpu

N_EMB = 256
D = 256
TILE = 1024


def _vq_kernel(x_ref, emb_ref, q_ref, perp_ref, hist_ref):
    i = pl.program_id(0)
    n_steps = pl.num_programs(0)

    @pl.when(i == 0)
    def _init():
        hist_ref[...] = jnp.zeros_like(hist_ref)

    f = x_ref[...]
    emb = emb_ref[...]
    sim = jnp.dot(f, emb, preferred_element_type=jnp.float32)
    row_norm = jnp.sum(f * f, axis=1, keepdims=True)
    emb_norm = jnp.sum(emb * emb, axis=0, keepdims=True)
    distances = row_norm + emb_norm - 2.0 * sim
    idx = jnp.argmin(distances, axis=1)
    onehot = (jax.lax.broadcasted_iota(jnp.int32, (TILE, N_EMB), 1)
              == idx[:, None]).astype(jnp.float32)
    q_ref[...] = jnp.dot(onehot, emb, preferred_element_type=jnp.float32)
    hist_ref[...] += jnp.sum(onehot, axis=0, keepdims=True)

    @pl.when(i == n_steps - 1)
    def _finish():
        total = jnp.float32(n_steps * TILE)
        avg_probs = hist_ref[...] / total
        ent = jnp.sum(avg_probs * jnp.log(avg_probs + 1e-10))
        perp_ref[...] = jnp.exp(-ent)[None, None]


@jax.jit
def kernel(x, embeddings):
    input_shape = x.shape
    flat = x.reshape(-1, D)
    n = flat.shape[0]
    grid = (n // TILE,)
    q, perp = pl.pallas_call(
        _vq_kernel,
        grid=grid,
        in_specs=[
            pl.BlockSpec((TILE, D), lambda i: (i, 0)),
            pl.BlockSpec((D, N_EMB), lambda i: (0, 0)),
        ],
        out_specs=[
            pl.BlockSpec((TILE, D), lambda i: (i, 0)),
            pl.BlockSpec((1, 1), lambda i: (0, 0)),
        ],
        out_shape=[
            jax.ShapeDtypeStruct((n, D), jnp.float32),
            jax.ShapeDtypeStruct((1, 1), jnp.float32),
        ],
        scratch_shapes=[pltpu.VMEM((1, N_EMB), jnp.float32)],
    )(flat, embeddings)
    return q.reshape(input_shape), perp[0, 0]
